# Initial kernel scaffold; baseline (speedup 1.0000x reference)
import jax
import jax.numpy as jnp
from jax.experimental import pallas as pl


def kernel(x, edge_index, W, b):
    raise NotImplementedError("write your pallas kernel here")


# speedup vs baseline: 19.3654x; 19.3654x over previous
import functools

import jax
import jax.numpy as jnp
from jax import lax
from jax.experimental import pallas as pl
from jax.experimental.pallas import tpu as pltpu
from jax.experimental.pallas import tpu_sc as plsc

N = 10000
D = 128
E = 320000

NC = 2
NS = 16
L = 16

N_PAD = 10240
ROWS_PER_TILE = N_PAD // NS
CH = 128
CHUNKS_PER_TILE = 79
E_TILE = CH * CHUNKS_PER_TILE
E_PAD = E_TILE * NC * NS
N_BLK = 1024
G_BLK = N_PAD // N_BLK

_mesh = plsc.VectorSubcoreMesh(core_axis_name="c", subcore_axis_name="s")


@functools.partial(
    pl.kernel,
    out_type=jax.ShapeDtypeStruct((NC, N_PAD), jnp.float32),
    mesh=_mesh,
    scratch_types=[
        pltpu.VMEM((CH,), jnp.int32),
        pltpu.VMEM((ROWS_PER_TILE,), jnp.float32),
        pltpu.VMEM_SHARED((N_PAD,), jnp.float32),
    ],
)
def _deg_call(col_hbm, out_hbm, idx_v, stage_v, deg_sh):
    c = lax.axis_index("c")
    s = lax.axis_index("s")

    def _zero(i, _):
        stage_v[pl.ds(i * L, L)] = jnp.zeros((L,), jnp.float32)
        return 0
    lax.fori_loop(0, ROWS_PER_TILE // L, _zero, 0)
    pltpu.sync_copy(stage_v, deg_sh.at[pl.ds(s * ROWS_PER_TILE, ROWS_PER_TILE)])

    def _ones(i, _):
        stage_v[pl.ds(i * L, L)] = jnp.full((L,), 1.0, jnp.float32)
        return 0
    lax.fori_loop(0, CH // L, _ones, 0)

    plsc.subcore_barrier()

    ebase = c * (E_PAD // NC) + s * E_TILE

    def _step(j, _):
        pltpu.sync_copy(col_hbm.at[pl.ds(ebase + j * CH, CH)], idx_v)
        pltpu.sync_copy(stage_v.at[pl.ds(0, CH)], deg_sh.at[idx_v], add=True)
        return 0
    lax.fori_loop(0, CHUNKS_PER_TILE, _step, 0)

    plsc.subcore_barrier()
    pltpu.sync_copy(
        deg_sh.at[pl.ds(s * ROWS_PER_TILE, ROWS_PER_TILE)],
        out_hbm.at[c, pl.ds(s * ROWS_PER_TILE, ROWS_PER_TILE)],
    )


@functools.partial(
    pl.kernel,
    out_type=jax.ShapeDtypeStruct((NC, N_PAD, D), jnp.float32),
    mesh=_mesh,
    scratch_types=[
        pltpu.VMEM((CH,), jnp.int32),
        pltpu.VMEM((CH,), jnp.int32),
        pltpu.VMEM((CH, D), jnp.float32),
        pltpu.VMEM_SHARED((N_PAD, D), jnp.float32),
        pltpu.SemaphoreType.DMA,
    ],
)
def _edge_call(g_hbm, row_hbm, col_hbm, out_hbm, ridx_v, cidx_v, rows_v, acc_sh, sem):
    c = lax.axis_index("c")
    s = lax.axis_index("s")

    def _zero(i, _):
        rows_v[i // (D // L), pl.ds((i % (D // L)) * L, L)] = jnp.zeros((L,), jnp.float32)
        return 0
    lax.fori_loop(0, CH * (D // L), _zero, 0)

    def _zchunk(k, _):
        pltpu.sync_copy(rows_v, acc_sh.at[pl.ds(s * ROWS_PER_TILE + k * CH, CH)])
        return 0
    lax.fori_loop(0, ROWS_PER_TILE // CH, _zchunk, 0)

    plsc.subcore_barrier()

    ebase = c * (E_PAD // NC) + s * E_TILE

    def _step(j, _):
        base = ebase + j * CH
        pltpu.sync_copy(row_hbm.at[pl.ds(base, CH)], ridx_v)
        pltpu.async_copy(g_hbm.at[ridx_v], rows_v, sem).wait()
        pltpu.sync_copy(col_hbm.at[pl.ds(base, CH)], cidx_v)
        pltpu.sync_copy(rows_v, acc_sh.at[cidx_v], add=True)
        return 0
    lax.fori_loop(0, CHUNKS_PER_TILE, _step, 0)

    plsc.subcore_barrier()

    def _wchunk(k, _):
        r0 = s * ROWS_PER_TILE + k * CH
        pltpu.sync_copy(acc_sh.at[pl.ds(r0, CH)], out_hbm.at[c, pl.ds(r0, CH)])
        return 0
    lax.fori_loop(0, ROWS_PER_TILE // CH, _wchunk, 0)


def _tc_transform_body(x_ref, w_ref, degp_ref, g_ref):
    deg = degp_ref[0] + degp_ref[1]
    dis = jnp.where(deg > 0, lax.rsqrt(deg), 0.0)
    h = jnp.dot(x_ref[...], w_ref[...], preferred_element_type=jnp.float32)
    g_ref[...] = (h.reshape(N_BLK // D, D, D) * dis[:, :, None]).reshape(N_BLK, D)


def _tc_transform(x_pad, W, degp3):
    return pl.pallas_call(
        _tc_transform_body,
        grid=(G_BLK,),
        in_specs=[
            pl.BlockSpec((N_BLK, D), lambda i: (i, 0)),
            pl.BlockSpec((D, D), lambda i: (0, 0)),
            pl.BlockSpec((NC, N_BLK // D, D), lambda i: (0, i, 0)),
        ],
        out_specs=pl.BlockSpec((N_BLK, D), lambda i: (i, 0)),
        out_shape=jax.ShapeDtypeStruct((N_PAD, D), jnp.float32),
    )(x_pad, W, degp3)


def _tc_final_body(p_ref, degp_ref, b_ref, o_ref):
    deg = degp_ref[0] + degp_ref[1]
    dis = jnp.where(deg > 0, lax.rsqrt(deg), 0.0)
    tot = p_ref[0] + p_ref[1]
    scaled = (tot.reshape(N_BLK // D, D, D) * dis[:, :, None]).reshape(N_BLK, D)
    o_ref[...] = scaled + b_ref[...]


def _tc_final(partials, degp3, b2):
    return pl.pallas_call(
        _tc_final_body,
        grid=(G_BLK,),
        in_specs=[
            pl.BlockSpec((NC, N_BLK, D), lambda i: (0, i, 0)),
            pl.BlockSpec((NC, N_BLK // D, D), lambda i: (0, i, 0)),
            pl.BlockSpec((1, D), lambda i: (0, 0)),
        ],
        out_specs=pl.BlockSpec((N_BLK, D), lambda i: (i, 0)),
        out_shape=jax.ShapeDtypeStruct((N_PAD, D), jnp.float32),
    )(partials, degp3, b2)


def kernel(x, edge_index, W, b):
    ei = edge_index.astype(jnp.int32)
    n_fill = E_PAD - E
    fill = N + (jnp.arange(n_fill, dtype=jnp.int32) % (N_PAD - N))
    row = jnp.concatenate([ei[0], fill])
    col = jnp.concatenate([ei[1], fill])
    x_pad = jnp.pad(x, ((0, N_PAD - N), (0, 0)))

    degp = _deg_call(col)
    degp3 = degp.reshape(NC, N_PAD // D, D)
    g = _tc_transform(x_pad, W, degp3)
    partials = _edge_call(g, row, col)
    out_pad = _tc_final(partials, degp3, b.reshape(1, D))
    return out_pad[:N]


# speedup vs baseline: 35.0830x; 1.8116x over previous
import functools

import jax
import jax.numpy as jnp
from jax import lax
from jax.experimental import pallas as pl
from jax.experimental.pallas import tpu as pltpu
from jax.experimental.pallas import tpu_sc as plsc

N = 10000
D = 128
E = 320000

NC = 2
NS = 16
L = 16

N_PAD = 10240
ROWS_PER_TILE = N_PAD // NS
CH = 128
CHUNKS_PER_TILE = 79
E_TILE = CH * CHUNKS_PER_TILE
E_PAD = E_TILE * NC * NS
N_BLK = 1024
G_BLK = N_PAD // N_BLK

_mesh = plsc.VectorSubcoreMesh(core_axis_name="c", subcore_axis_name="s")


@functools.partial(
    pl.kernel,
    out_type=jax.ShapeDtypeStruct((NC, N_PAD), jnp.float32),
    mesh=_mesh,
    scratch_types=[
        pltpu.VMEM((CHUNKS_PER_TILE, CH), jnp.int32),
        pltpu.VMEM((ROWS_PER_TILE,), jnp.float32),
        pltpu.VMEM_SHARED((N_PAD,), jnp.float32),
        pltpu.SemaphoreType.DMA,
    ],
)
def _deg_call(col3_hbm, out_hbm, idx_v, stage_v, deg_sh, sem):
    c = lax.axis_index("c")
    s = lax.axis_index("s")
    wid = c * NS + s

    idx_cp = pltpu.async_copy(col3_hbm.at[wid], idx_v, sem)

    def _zero(i, _):
        stage_v[pl.ds(i * L, L)] = jnp.zeros((L,), jnp.float32)
        return 0
    lax.fori_loop(0, ROWS_PER_TILE // L, _zero, 0)
    pltpu.sync_copy(stage_v, deg_sh.at[pl.ds(s * ROWS_PER_TILE, ROWS_PER_TILE)])

    def _ones(i, _):
        stage_v[pl.ds(i * L, L)] = jnp.full((L,), 1.0, jnp.float32)
        return 0
    lax.fori_loop(0, CH // L, _ones, 0)

    idx_cp.wait()
    plsc.subcore_barrier()

    def _step(j, _):
        pltpu.async_copy(stage_v.at[pl.ds(0, CH)], deg_sh.at[idx_v.at[j]], sem,
                         add=True)
        return 0
    lax.fori_loop(0, CHUNKS_PER_TILE, _step, 0)

    def _drain(j, _):
        pltpu.make_async_copy(stage_v.at[pl.ds(0, CH)], deg_sh.at[idx_v.at[0]],
                              sem).wait()
        return 0
    lax.fori_loop(0, CHUNKS_PER_TILE, _drain, 0)

    plsc.subcore_barrier()
    pltpu.sync_copy(
        deg_sh.at[pl.ds(s * ROWS_PER_TILE, ROWS_PER_TILE)],
        out_hbm.at[c, pl.ds(s * ROWS_PER_TILE, ROWS_PER_TILE)],
    )


@functools.partial(
    pl.kernel,
    out_type=jax.ShapeDtypeStruct((NC, N_PAD, D), jnp.float32),
    mesh=_mesh,
    scratch_types=[
        pltpu.VMEM((3, CH), jnp.int32),
        pltpu.VMEM((3, CH), jnp.int32),
        pltpu.VMEM((2, CH, D), jnp.float32),
        pltpu.VMEM_SHARED((N_PAD, D), jnp.float32),
        pltpu.SemaphoreType.DMA,
        pltpu.SemaphoreType.DMA,
        pltpu.SemaphoreType.DMA,
    ],
)
def _edge_call(g_hbm, row3_hbm, col3_hbm, out_hbm, ridx_v, cidx_v, rows_v,
               acc_sh, gsem, ssem, isem):
    c = lax.axis_index("c")
    s = lax.axis_index("s")
    wid = c * NS + s
    last = CHUNKS_PER_TILE - 1

    pltpu.sync_copy(row3_hbm.at[wid, 0], ridx_v.at[0])
    pltpu.sync_copy(col3_hbm.at[wid, 0], cidx_v.at[0])
    pltpu.async_copy(row3_hbm.at[wid, 1], ridx_v.at[1], isem)
    pltpu.async_copy(col3_hbm.at[wid, 1], cidx_v.at[1], isem)

    def _zero(i, _):
        rows_v[0, i // (D // L), pl.ds((i % (D // L)) * L, L)] = (
            jnp.zeros((L,), jnp.float32))
        return 0
    lax.fori_loop(0, CH * (D // L), _zero, 0)

    def _zchunk(k, _):
        pltpu.sync_copy(rows_v.at[0], acc_sh.at[pl.ds(s * ROWS_PER_TILE + k * CH, CH)])
        return 0
    lax.fori_loop(0, ROWS_PER_TILE // CH, _zchunk, 0)

    plsc.subcore_barrier()

    pltpu.async_copy(g_hbm.at[ridx_v.at[0]], rows_v.at[0], gsem)

    def _step(j, _):
        buf = jax.lax.rem(j, 2)
        tj = jax.lax.rem(j, 3)
        tj1 = jax.lax.rem(j + 1, 3)
        tj2 = jax.lax.rem(j + 2, 3)
        pltpu.make_async_copy(g_hbm.at[ridx_v.at[tj]], rows_v.at[buf], gsem).wait()

        @pl.when(j > 0)
        def _():
            pltpu.make_async_copy(rows_v.at[1 - buf],
                                  acc_sh.at[cidx_v.at[tj]], ssem).wait()

        @pl.when(j + 2 <= last)
        def _():
            pltpu.async_copy(row3_hbm.at[wid, j + 2], ridx_v.at[tj2], isem)
            pltpu.async_copy(col3_hbm.at[wid, j + 2], cidx_v.at[tj2], isem)

        @pl.when(j + 1 <= last)
        def _():
            pltpu.make_async_copy(row3_hbm.at[wid, 0], ridx_v.at[tj1], isem).wait()
            pltpu.make_async_copy(col3_hbm.at[wid, 0], cidx_v.at[tj1], isem).wait()
            pltpu.async_copy(g_hbm.at[ridx_v.at[tj1]], rows_v.at[1 - buf], gsem)

        pltpu.async_copy(rows_v.at[buf], acc_sh.at[cidx_v.at[tj]], ssem, add=True)
        return 0
    lax.fori_loop(0, CHUNKS_PER_TILE, _step, 0)
    pltpu.make_async_copy(rows_v.at[0], acc_sh.at[cidx_v.at[0]], ssem).wait()

    plsc.subcore_barrier()

    def _wchunk(k, _):
        r0 = s * ROWS_PER_TILE + k * CH
        pltpu.sync_copy(acc_sh.at[pl.ds(r0, CH)], out_hbm.at[c, pl.ds(r0, CH)])
        return 0
    lax.fori_loop(0, ROWS_PER_TILE // CH, _wchunk, 0)


def _tc_transform_body(x_ref, w_ref, degp_ref, g_ref):
    deg = degp_ref[0] + degp_ref[1]
    dis = jnp.where(deg > 0, lax.rsqrt(deg), 0.0)
    h = jnp.dot(x_ref[...], w_ref[...], preferred_element_type=jnp.float32)
    g_ref[...] = (h.reshape(N_BLK // D, D, D) * dis[:, :, None]).reshape(N_BLK, D)


def _tc_transform(x_pad, W, degp3):
    return pl.pallas_call(
        _tc_transform_body,
        grid=(G_BLK,),
        in_specs=[
            pl.BlockSpec((N_BLK, D), lambda i: (i, 0)),
            pl.BlockSpec((D, D), lambda i: (0, 0)),
            pl.BlockSpec((NC, N_BLK // D, D), lambda i: (0, i, 0)),
        ],
        out_specs=pl.BlockSpec((N_BLK, D), lambda i: (i, 0)),
        out_shape=jax.ShapeDtypeStruct((N_PAD, D), jnp.float32),
    )(x_pad, W, degp3)


def _tc_final_body(p_ref, degp_ref, b_ref, o_ref):
    deg = degp_ref[0] + degp_ref[1]
    dis = jnp.where(deg > 0, lax.rsqrt(deg), 0.0)
    tot = p_ref[0] + p_ref[1]
    scaled = (tot.reshape(N_BLK // D, D, D) * dis[:, :, None]).reshape(N_BLK, D)
    o_ref[...] = scaled + b_ref[...]


def _tc_final(partials, degp3, b2):
    return pl.pallas_call(
        _tc_final_body,
        grid=(G_BLK,),
        in_specs=[
            pl.BlockSpec((NC, N_BLK, D), lambda i: (0, i, 0)),
            pl.BlockSpec((NC, N_BLK // D, D), lambda i: (0, i, 0)),
            pl.BlockSpec((1, D), lambda i: (0, 0)),
        ],
        out_specs=pl.BlockSpec((N_BLK, D), lambda i: (i, 0)),
        out_shape=jax.ShapeDtypeStruct((N_PAD, D), jnp.float32),
    )(partials, degp3, b2)


def kernel(x, edge_index, W, b):
    ei = edge_index.astype(jnp.int32)
    n_fill = E_PAD - E
    fill = N + (jnp.arange(n_fill, dtype=jnp.int32) % (N_PAD - N))
    row3 = jnp.concatenate([ei[0], fill]).reshape(NC * NS, CHUNKS_PER_TILE, CH)
    col3 = jnp.concatenate([ei[1], fill]).reshape(NC * NS, CHUNKS_PER_TILE, CH)
    x_pad = jnp.pad(x, ((0, N_PAD - N), (0, 0)))

    degp = _deg_call(col3)
    degp3 = degp.reshape(NC, N_PAD // D, D)
    g = _tc_transform(x_pad, W, degp3)
    partials = _edge_call(g, row3, col3)
    out_pad = _tc_final(partials, degp3, b.reshape(1, D))
    return out_pad[:N]
